# baseline (speedup 1.0000x reference)
import functools

import jax
import jax.numpy as jnp
from jax import lax
from jax.experimental import pallas as pl
from jax.experimental.pallas import tpu as pltpu
from jax.experimental.pallas import tpu_sc as plsc

NUM_GLYPHS = 5976
LUT_PAD = 6016
ENT_PAD = 2048
NGRP = 13
D = 64
NC, NS = 2, 16
NW = NC * NS
CH = 128
N_TOTAL = 1024 * 200
NPW = N_TOTAL // NW
NCH = NPW // CH


def _prep_body(ent_ref, grp_ref, out_ref):
    out_ref[...] = ent_ref[...] + grp_ref[0]


_prep = pl.pallas_call(
    _prep_body,
    grid=(NGRP,),
    in_specs=[
        pl.BlockSpec((ENT_PAD // 2, 2 * D), lambda j: (0, 0)),
        pl.BlockSpec((1, 1, 2 * D), lambda j: (j, 0, 0)),
    ],
    out_specs=pl.BlockSpec((ENT_PAD // 2, 2 * D), lambda j: (j, 0)),
    out_shape=jax.ShapeDtypeStruct((NGRP * ENT_PAD // 2, 2 * D),
                                   jnp.float32),
)


def _make_lookup():
    mesh = plsc.VectorSubcoreMesh(
        core_axis_name="c", subcore_axis_name="s",
        num_cores=NC, num_subcores=NS)

    @functools.partial(
        pl.kernel, mesh=mesh,
        compiler_params=pltpu.CompilerParams(
            needs_layout_passes=False, use_tc_tiling_on_sc=False),
        out_type=jax.ShapeDtypeStruct((N_TOTAL // CH, CH, D), jnp.float32),
        scratch_types=[
            pltpu.VMEM((NPW,), jnp.int32),
            pltpu.VMEM((LUT_PAD,), jnp.int32),
            pltpu.VMEM((LUT_PAD,), jnp.int32),
            pltpu.VMEM((6, CH), jnp.int32),
            pltpu.VMEM((6, CH, D), jnp.float32),
            pltpu.SemaphoreType.DMA,
            pltpu.SemaphoreType.DMA,
            pltpu.SemaphoreType.DMA,
            pltpu.SemaphoreType.DMA,
            pltpu.SemaphoreType.DMA,
            pltpu.SemaphoreType.DMA,
            pltpu.SemaphoreType.DMA,
            pltpu.SemaphoreType.DMA,
            pltpu.SemaphoreType.DMA,
            pltpu.SemaphoreType.DMA,
            pltpu.SemaphoreType.DMA,
            pltpu.SemaphoreType.DMA,
        ],
    )
    def lookup(ct_hbm, elut_hbm, glut_hbm, gl_hbm, out_hbm,
               gl_v, elut_v, glut_v, idx_v, rows_v,
               gsem0, gsem1, gsem2, gsem3, gsem4, gsem5,
               wsem0, wsem1, wsem2, wsem3, wsem4, wsem5):
        wid = lax.axis_index("s") * NC + lax.axis_index("c")
        base = pl.multiple_of(wid * NPW, NPW)
        kbase = pl.multiple_of(wid * NCH, NCH)
        pltpu.sync_copy(gl_hbm.at[pl.ds(base, NPW)], gl_v)
        pltpu.sync_copy(elut_hbm, elut_v)
        pltpu.sync_copy(glut_hbm, glut_v)
        gsems = (gsem0, gsem1, gsem2, gsem3, gsem4, gsem5)
        wsems = (wsem0, wsem1, wsem2, wsem3, wsem4, wsem5)
        nslot = 6
        look = 3

        def indices(j, slot):
            off = pl.multiple_of(j * CH, CH)
            for t in range(CH // 16):
                g = gl_v[pl.ds(off + t * 16, 16)]
                ge = plsc.load_gather(elut_v, [g])
                gg = plsc.load_gather(glut_v, [g])
                idx_v[slot, pl.ds(t * 16, 16)] = gg * ENT_PAD + ge

        def gather_start(slot):
            pltpu.async_copy(ct_hbm.at[idx_v.at[slot]], rows_v.at[slot],
                             gsems[slot])

        def gather_wait(slot):
            pltpu.make_async_copy(ct_hbm.at[idx_v.at[slot]],
                                  rows_v.at[slot], gsems[slot]).wait()

        def write_start(slot, j):
            pltpu.async_copy(rows_v.at[slot], out_hbm.at[kbase + j],
                             wsems[slot])

        def write_wait(slot, j):
            pltpu.make_async_copy(rows_v.at[slot], out_hbm.at[kbase + j],
                                  wsems[slot]).wait()

        for j in range(look):
            indices(j, j % nslot)
            gather_start(j % nslot)
        for j in range(NCH):
            jn = j + look
            if jn < NCH:
                sn = jn % nslot
                indices(jn, sn)
                if jn - nslot >= 0:
                    write_wait(sn, jn - nslot)
                gather_start(sn)
            gather_wait(j % nslot)
            write_start(j % nslot, j)
        for j in range(max(0, NCH - nslot), NCH):
            write_wait(j % nslot, j)

    return lookup


_lookup = _make_lookup()


def kernel(glyphs, entity_lut, group_lut, entity_table, group_table):
    b, s = glyphs.shape
    gl = glyphs.astype(jnp.int32).reshape(b * s)
    elut = jnp.pad(entity_lut.astype(jnp.int32), (0, LUT_PAD - NUM_GLYPHS))
    glut = jnp.pad(group_lut.astype(jnp.int32), (0, LUT_PAD - NUM_GLYPHS))
    ent_p = jnp.pad(entity_table,
                    ((0, ENT_PAD - entity_table.shape[0]), (0, 0)))
    ent_p2 = ent_p.reshape(ENT_PAD // 2, 2 * D)
    grp3 = jnp.concatenate([group_table, group_table],
                           axis=1).reshape(NGRP, 1, 2 * D)
    ctable = _prep(ent_p2, grp3).reshape(NGRP * ENT_PAD, D)
    out = _lookup(ctable, elut, glut, gl)
    return out.reshape(b, s, D)
